# baseline (speedup 1.0000x reference)
import functools

import jax
import jax.numpy as jnp
from jax import lax
from jax.experimental import pallas as pl
from jax.experimental.pallas import tpu as pltpu
from jax.experimental.pallas import tpu_sc as plsc

_V_DIAG = 100000
_V_PRESC = 1000
_V_YDELSE = 1000
_V_TOT = _V_DIAG + _V_PRESC + _V_YDELSE
_PAD_TOK = _V_TOT
_TBL = _V_TOT + 8
_BATCH = 1024
_NTOK = 300
_CHUNK = 128
_CPR = 3
_TPR = _CHUNK * _CPR
_NWORKERS = 32
_RPW = _BATCH // _NWORKERS
_TPW = _RPW * _TPR

_mesh = plsc.VectorSubcoreMesh(core_axis_name="c", subcore_axis_name="s")


@functools.partial(
    pl.kernel,
    out_type=jax.ShapeDtypeStruct((_BATCH * 16,), jnp.float32),
    mesh=_mesh,
    compiler_params=pltpu.CompilerParams(
        needs_layout_passes=False, use_tc_tiling_on_sc=False
    ),
    scratch_types=[
        pltpu.VMEM((_TPW,), jnp.int32),
        pltpu.VMEM((_TBL,), jnp.int32),
        pltpu.VMEM((_TPW,), jnp.int32),
        pltpu.VMEM((_RPW * 16,), jnp.float32),
        pltpu.SemaphoreType.DMA,
    ],
)
def _bow_sc(tok_hbm, w_hbm, out_hbm, tok_v, mark, g, out_v, sem):
    wid = lax.axis_index("s") * 2 + lax.axis_index("c")
    pltpu.sync_copy(tok_hbm.at[pl.ds(wid * _TPW, _TPW)], tok_v)

    lanes = lax.iota(jnp.int32, 16)

    pltpu.async_copy(w_hbm.at[tok_v], g, sem)

    def scatter_row(r):
        for c in range(_CPR):
            for v in range(_CHUNK // 16):
                tv = tok_v[pl.ds(r * _TPR + c * _CHUNK + v * 16, 16)]
                jv = lanes + (c * _CHUNK + v * 16)
                plsc.store_scatter(mark, [tv], jv)

    scatter_row(0)
    pltpu.make_async_copy(w_hbm.at[tok_v], g, sem).wait()

    def row_body(r, carry):
        acc0 = jnp.zeros((16,), jnp.float32)
        acc1 = jnp.zeros((16,), jnp.float32)
        for c in range(_CPR):
            for v in range(_CHUNK // 16):
                off = r * _TPR + c * _CHUNK + v * 16
                tv = tok_v[pl.ds(off, 16)]
                jv = lanes + (c * _CHUNK + v * 16)
                keep = plsc.load_gather(mark, [tv]) == jv
                pw = g[pl.ds(off, 16)]
                w0v = lax.bitcast_convert_type(
                    lax.shift_left(pw, 16), jnp.float32
                )
                w1v = lax.bitcast_convert_type(
                    lax.bitwise_and(pw, jnp.int32(-65536)), jnp.float32
                )
                acc0 = acc0 + jnp.where(keep, w0v, 0.0)
                acc1 = acc1 + jnp.where(keep, w1v, 0.0)
        s0 = jnp.sum(acc0)
        s1 = jnp.sum(acc1)
        res = jnp.where(lanes == 0, s0, jnp.where(lanes == 1, s1, 0.0))
        out_v[pl.ds(r * 16, 16)] = res

        @pl.when(r < _RPW - 1)
        def _():
            scatter_row(r + 1)

        return carry

    lax.fori_loop(0, _RPW, row_body, 0)
    pltpu.sync_copy(out_v, out_hbm.at[pl.ds(wid * _RPW * 16, _RPW * 16)])


def kernel(diag_tokens, prescription_tokens, ydelse_tokens, W, b):
    tok = jnp.concatenate(
        [
            diag_tokens.astype(jnp.int32),
            prescription_tokens.astype(jnp.int32) + _V_DIAG,
            ydelse_tokens.astype(jnp.int32) + (_V_DIAG + _V_PRESC),
        ],
        axis=1,
    )
    tok = jnp.pad(tok, ((0, 0), (0, _TPR - _NTOK)), constant_values=_PAD_TOK)
    tok = tok.reshape(_BATCH * _TPR)
    wfull = jnp.concatenate(
        [W, b[None, :], jnp.zeros((_TBL - _V_TOT - 1, 2), jnp.float32)], axis=0
    )
    wb = jax.lax.bitcast_convert_type(
        wfull.astype(jnp.bfloat16), jnp.uint16
    ).astype(jnp.uint32)
    w01 = (wb[:, 0] | (wb[:, 1] << 16)).astype(jnp.int32)
    out = _bow_sc(tok, w01)
    return out.reshape(_BATCH, 16)[:, :2]


# speedup vs baseline: 11.5457x; 11.5457x over previous
import functools

import jax
import jax.numpy as jnp
from jax import lax
from jax.experimental import pallas as pl
from jax.experimental.pallas import tpu as pltpu
from jax.experimental.pallas import tpu_sc as plsc

_V_DIAG = 100000
_V_PRESC = 1000
_V_YDELSE = 1000
_V_TOT = _V_DIAG + _V_PRESC + _V_YDELSE
_PAD_TOK = _V_TOT
_TBL = _V_TOT + 8
_BATCH = 1024
_NTOK = 300
_TPR = 304
_VPR = _TPR // 16
_NWORKERS = 32
_RPW = _BATCH // _NWORKERS
_TPW = _RPW * _TPR

_mesh = plsc.VectorSubcoreMesh(core_axis_name="c", subcore_axis_name="s")


@functools.partial(
    pl.kernel,
    out_type=jax.ShapeDtypeStruct((_BATCH * 16,), jnp.float32),
    mesh=_mesh,
    compiler_params=pltpu.CompilerParams(
        needs_layout_passes=False, use_tc_tiling_on_sc=False
    ),
    scratch_types=[
        pltpu.VMEM((_TPW,), jnp.int32),
        pltpu.VMEM((_TBL,), jnp.int32),
        pltpu.VMEM((_TPW,), jnp.int32),
        pltpu.VMEM((_RPW * 16,), jnp.float32),
        pltpu.VMEM_SHARED((_TBL,), jnp.int32),
        pltpu.SemaphoreType.DMA,
    ],
)
def _bow_sc(tok_hbm, w_hbm, out_hbm, tok_v, mark, g, out_v, w_sh, sem):
    sid = lax.axis_index("s")
    wid = sid * 2 + lax.axis_index("c")
    pltpu.sync_copy(tok_hbm.at[pl.ds(wid * _TPW, _TPW)], tok_v)

    lanes = lax.iota(jnp.int32, 16)

    @pl.when(sid == 0)
    def _():
        pltpu.sync_copy(w_hbm, w_sh)

    plsc.subcore_barrier()

    pltpu.async_copy(w_sh.at[tok_v], g, sem)

    def scatter_row(r):
        for v in range(_VPR):
            tv = tok_v[pl.ds(r * _TPR + v * 16, 16)]
            plsc.store_scatter(mark, [tv], lanes + v * 16)

    scatter_row(0)
    pltpu.make_async_copy(w_sh.at[tok_v], g, sem).wait()

    def row_body(r, carry):
        acc0 = jnp.zeros((16,), jnp.float32)
        acc1 = jnp.zeros((16,), jnp.float32)
        for v in range(_VPR):
            off = r * _TPR + v * 16
            tv = tok_v[pl.ds(off, 16)]
            keep = plsc.load_gather(mark, [tv]) == lanes + v * 16
            pw = g[pl.ds(off, 16)]
            w0v = lax.bitcast_convert_type(lax.shift_left(pw, 16), jnp.float32)
            w1v = lax.bitcast_convert_type(
                lax.bitwise_and(pw, jnp.int32(-65536)), jnp.float32
            )
            acc0 = acc0 + jnp.where(keep, w0v, 0.0)
            acc1 = acc1 + jnp.where(keep, w1v, 0.0)
        s0 = jnp.sum(acc0)
        s1 = jnp.sum(acc1)
        res = jnp.where(lanes == 0, s0, jnp.where(lanes == 1, s1, 0.0))
        out_v[pl.ds(r * 16, 16)] = res

        @pl.when(r < _RPW - 1)
        def _():
            scatter_row(r + 1)

        return carry

    lax.fori_loop(0, _RPW, row_body, 0)
    pltpu.sync_copy(out_v, out_hbm.at[pl.ds(wid * _RPW * 16, _RPW * 16)])


def kernel(diag_tokens, prescription_tokens, ydelse_tokens, W, b):
    tok = jnp.concatenate(
        [
            diag_tokens.astype(jnp.int32),
            prescription_tokens.astype(jnp.int32) + _V_DIAG,
            ydelse_tokens.astype(jnp.int32) + (_V_DIAG + _V_PRESC),
        ],
        axis=1,
    )
    tok = jnp.pad(tok, ((0, 0), (0, _TPR - _NTOK)), constant_values=_PAD_TOK)
    tok = tok.reshape(_BATCH * _TPR)
    wfull = jnp.concatenate(
        [W, b[None, :], jnp.zeros((_TBL - _V_TOT - 1, 2), jnp.float32)], axis=0
    )
    wb = jax.lax.bitcast_convert_type(
        wfull.astype(jnp.bfloat16), jnp.uint16
    ).astype(jnp.uint32)
    w01 = (wb[:, 0] | (wb[:, 1] << 16)).astype(jnp.int32)
    out = _bow_sc(tok, w01)
    return out.reshape(_BATCH, 16)[:, :2]
